# baseline (speedup 1.0000x reference)
import functools

import jax
import jax.numpy as jnp
from jax import lax
from jax.experimental import pallas as pl
from jax.experimental.pallas import tpu as pltpu
from jax.experimental.pallas import tpu_sc as plsc

N_ROWS = 100000
N_FEAT = 256
N_SEG = 128
LANES = 16
NFC = N_FEAT // LANES
N_CORES = 2
N_SUBCORES = 16
NW = N_CORES * N_SUBCORES

RPT = 3136
LAST_ROWS = N_ROWS - (NW - 1) * RPT
CHUNK = 32
FULL_CHUNKS = RPT // CHUNK
LAST_CHUNKS = LAST_ROWS // CHUNK

NBUF = 12
NSTAGE = 8

NEG_INF = -float("inf")


def _sc_pool_body(x_hbm, batch_hbm, psum_hbm, pmax_hbm, pcnt_hbm,
                  xbuf, bbuf, run, stage, acc_c, sem, fsem):
    wid = lax.axis_index("s") * N_CORES + lax.axis_index("c")
    rbase = wid * RPT
    is_last = wid == NW - 1
    nchunks = jnp.where(is_last, LAST_CHUNKS, FULL_CHUNKS)
    nrows = jnp.where(is_last, LAST_ROWS, RPT)

    zeros = jnp.zeros((LANES,), jnp.float32)
    ninf = jnp.full((LANES,), NEG_INF, jnp.float32)

    def start_x_copy(k, b):
        pltpu.async_copy(x_hbm.at[pl.ds(rbase + k * CHUNK, CHUNK), :],
                         xbuf.at[b], sem)

    def wait_x_copy():
        pltpu.make_async_copy(x_hbm.at[pl.ds(0, CHUNK), :], xbuf.at[0],
                              sem).wait()

    for d in range(NBUF):
        @pl.when(d < nchunks)
        def _():
            start_x_copy(d, d)

    def copy_batch_full(_):
        pltpu.sync_copy(batch_hbm.at[pl.ds(rbase, RPT)], bbuf.at[pl.ds(0, RPT)])
        return 0

    def copy_batch_last(_):
        pltpu.sync_copy(batch_hbm.at[pl.ds(rbase, LAST_ROWS)],
                        bbuf.at[pl.ds(0, LAST_ROWS)])
        return 0

    lax.cond(is_last, copy_batch_last, copy_batch_full, 0)

    def init_cnt(r, _):
        acc_c[r, :] = zeros
        return 0

    lax.fori_loop(0, N_SEG, init_cnt, 0)

    id_carry = (zeros,) * NFC + (ninf,) * NFC

    def reset_run():
        for j in range(NFC):
            sl = pl.ds(j * LANES, LANES)
            run[0, sl] = zeros
            run[1, sl] = ninf

    def flush(cur_seg, cnt, fp):
        p = lax.rem(fp, NSTAGE)
        for j in range(NFC):
            sl = pl.ds(j * LANES, LANES)
            stage[p, 0, sl] = run[0, sl]
            stage[p, 1, sl] = run[1, sl]
        acc_c[cur_seg, :] = jnp.broadcast_to(cnt, (LANES,))
        pltpu.sync_copy(stage.at[p, 0], psum_hbm.at[wid, cur_seg])
        pltpu.sync_copy(stage.at[p, 1], pmax_hbm.at[wid, cur_seg])

    def process_chunk(k, carry):
        cur_seg, cnt, fp = carry
        wait_x_copy()
        b = lax.rem(k, NBUF)
        t0 = k * CHUNK
        seg0 = bbuf[pl.ds(t0, LANES)][0]
        seg_last = bbuf[pl.ds(t0 + CHUNK - LANES, LANES)][LANES - 1]

        def fast(c):
            cur_seg, cnt, fp = c
            is_new = seg0 != cur_seg

            @pl.when(is_new)
            def _():
                flush(cur_seg, cnt, fp)
                reset_run()

            fp = fp + jnp.where(is_new, 1, 0).astype(jnp.int32)
            cnt = jnp.where(is_new, jnp.float32(0.0), cnt)

            def rbody(r, rc):
                out = []
                for j in range(NFC):
                    v = xbuf[b, r, pl.ds(j * LANES, LANES)]
                    out.append(rc[j] + v)
                for j in range(NFC):
                    v = xbuf[b, r, pl.ds(j * LANES, LANES)]
                    out.append(jnp.maximum(rc[NFC + j], v))
                return tuple(out)

            regs = lax.fori_loop(0, CHUNK, rbody, id_carry)
            for j in range(NFC):
                sl = pl.ds(j * LANES, LANES)
                plsc.addupdate(run.at[0, sl], regs[j])
                run[1, sl] = jnp.maximum(run[1, sl], regs[NFC + j])
            return (seg0, cnt + float(CHUNK), fp)

        def slow(c):
            def do_row(i, c):
                cur_seg, cnt, fp = c
                seg_i = bbuf[pl.ds(t0 + i, LANES)][0]
                is_new = seg_i != cur_seg

                @pl.when(is_new)
                def _():
                    flush(cur_seg, cnt, fp)
                    reset_run()

                fp = fp + jnp.where(is_new, 1, 0).astype(jnp.int32)
                cnt = jnp.where(is_new, jnp.float32(0.0), cnt)
                for j in range(NFC):
                    sl = pl.ds(j * LANES, LANES)
                    v = xbuf[b, i, sl]
                    plsc.addupdate(run.at[0, sl], v)
                    run[1, sl] = jnp.maximum(run[1, sl], v)
                return (seg_i, cnt + 1.0, fp)

            return lax.fori_loop(0, CHUNK, do_row, c)

        res = lax.cond(seg0 == seg_last, fast, slow, (cur_seg, cnt, fp))

        @pl.when(k + NBUF < nchunks)
        def _():
            start_x_copy(k + NBUF, b)

        return res

    reset_run()
    carry0 = (bbuf[pl.ds(0, LANES)][0], jnp.float32(0.0), jnp.int32(0))
    carry = lax.fori_loop(0, nchunks, process_chunk, carry0)

    cur_seg, cnt, fp = carry
    flush(cur_seg, cnt, fp)
    pltpu.sync_copy(acc_c, pcnt_hbm.at[wid])


@functools.partial(
    pl.kernel,
    out_type=(
        jax.ShapeDtypeStruct((NW, N_SEG, N_FEAT), jnp.float32),
        jax.ShapeDtypeStruct((NW, N_SEG, N_FEAT), jnp.float32),
        jax.ShapeDtypeStruct((NW, N_SEG, LANES), jnp.float32),
    ),
    mesh=plsc.VectorSubcoreMesh(core_axis_name="c", subcore_axis_name="s"),
    scratch_types=[
        pltpu.VMEM((NBUF, CHUNK, N_FEAT), jnp.float32),
        pltpu.VMEM((RPT + 3 * LANES,), jnp.int32),
        pltpu.VMEM((2, N_FEAT), jnp.float32),
        pltpu.VMEM((NSTAGE, 2, N_FEAT), jnp.float32),
        pltpu.VMEM((N_SEG, LANES), jnp.float32),
        pltpu.SemaphoreType.DMA,
        pltpu.SemaphoreType.DMA,
    ],
)
def _sc_pool(x_hbm, batch_hbm, psum_hbm, pmax_hbm, pcnt_hbm,
             xbuf, bbuf, run, stage, acc_c, sem, fsem):
    _sc_pool_body(x_hbm, batch_hbm, psum_hbm, pmax_hbm, pcnt_hbm,
                  xbuf, bbuf, run, stage, acc_c, sem, fsem)


def _tc_merge_body(ps_ref, pm_ref, pc_ref, out_ref):
    valid = pc_ref[...][:, :, 0:1] > 0.0
    ps = jnp.where(valid, ps_ref[...], jnp.float32(0.0))
    pm = jnp.where(valid, pm_ref[...], NEG_INF)
    s = jnp.sum(ps, axis=0)
    m = jnp.max(pm, axis=0)
    c = jnp.sum(pc_ref[...], axis=0)[:, 0:1]
    mean = s / jnp.maximum(c, 1.0)
    mx = jnp.where(m == NEG_INF, jnp.float32(0.0), m)
    out_ref[...] = jnp.concatenate([mean, mx], axis=-1)


def _tc_merge(psum, pmax, pcnt):
    return pl.pallas_call(
        _tc_merge_body,
        out_shape=jax.ShapeDtypeStruct((N_SEG, 2 * N_FEAT), jnp.float32),
    )(psum, pmax, pcnt)


@jax.jit
def kernel(x, batch):
    batch32 = batch.astype(jnp.int32)
    psum, pmax, pcnt = _sc_pool(x, batch32)
    return _tc_merge(psum, pmax, pcnt)


# speedup vs baseline: 1.0071x; 1.0071x over previous
import functools

import jax
import jax.numpy as jnp
from jax import lax
from jax.experimental import pallas as pl
from jax.experimental.pallas import tpu as pltpu
from jax.experimental.pallas import tpu_sc as plsc

N_ROWS = 100000
N_FEAT = 256
N_SEG = 128
LANES = 16
NFC = N_FEAT // LANES
N_CORES = 2
N_SUBCORES = 16
NW = N_CORES * N_SUBCORES

RPT = 3136
LAST_ROWS = N_ROWS - (NW - 1) * RPT
CHUNK = 32
FULL_CHUNKS = RPT // CHUNK
LAST_CHUNKS = LAST_ROWS // CHUNK

NBUF = 12
NSTAGE = 8

NEG_INF = -float("inf")


def _sc_pool_body(x_hbm, batch_hbm, psum_hbm, pmax_hbm, pcnt_hbm,
                  xbuf, bbuf, run, stage, acc_c, sem, fsem):
    wid = lax.axis_index("s") * N_CORES + lax.axis_index("c")
    rbase = wid * RPT
    is_last = wid == NW - 1
    nchunks = jnp.where(is_last, LAST_CHUNKS, FULL_CHUNKS)
    nrows = jnp.where(is_last, LAST_ROWS, RPT)

    zeros = jnp.zeros((LANES,), jnp.float32)
    ninf = jnp.full((LANES,), NEG_INF, jnp.float32)

    def start_x_copy(k, b):
        pltpu.async_copy(x_hbm.at[pl.ds(rbase + k * CHUNK, CHUNK), :],
                         xbuf.at[b], sem)

    def wait_x_copy():
        pltpu.make_async_copy(x_hbm.at[pl.ds(0, CHUNK), :], xbuf.at[0],
                              sem).wait()

    for d in range(NBUF):
        @pl.when(d < nchunks)
        def _():
            start_x_copy(d, d)

    def copy_batch_full(_):
        pltpu.sync_copy(batch_hbm.at[pl.ds(rbase, RPT)], bbuf.at[pl.ds(0, RPT)])
        return 0

    def copy_batch_last(_):
        pltpu.sync_copy(batch_hbm.at[pl.ds(rbase, LAST_ROWS)],
                        bbuf.at[pl.ds(0, LAST_ROWS)])
        return 0

    lax.cond(is_last, copy_batch_last, copy_batch_full, 0)

    def init_cnt(r, _):
        acc_c[r, :] = zeros
        return 0

    lax.fori_loop(0, N_SEG, init_cnt, 0)

    id_carry = (zeros,) * NFC + (ninf,) * NFC

    def reset_run():
        for j in range(NFC):
            sl = pl.ds(j * LANES, LANES)
            run[0, sl] = zeros
            run[1, sl] = ninf

    def flush(cur_seg, cnt, fp):
        @pl.when(fp >= NSTAGE)
        def _():
            pltpu.make_async_copy(psum_hbm.at[0, pl.ds(0, 2), :],
                                  stage.at[0], fsem).wait()

        p = lax.rem(fp, NSTAGE)
        for j in range(NFC):
            sl = pl.ds(j * LANES, LANES)
            stage[p, 0, sl] = run[0, sl]
            stage[p, 1, sl] = run[1, sl]
        acc_c[cur_seg, :] = jnp.broadcast_to(cnt, (LANES,))
        pltpu.async_copy(stage.at[p, 0], psum_hbm.at[wid, cur_seg], fsem)
        pltpu.async_copy(stage.at[p, 1], pmax_hbm.at[wid, cur_seg], fsem)

    def process_chunk(k, carry):
        cur_seg, cnt, fp = carry
        wait_x_copy()
        b = lax.rem(k, NBUF)
        t0 = k * CHUNK
        seg0 = bbuf[pl.ds(t0, LANES)][0]
        seg_last = bbuf[pl.ds(t0 + CHUNK - LANES, LANES)][LANES - 1]

        def fast(c):
            cur_seg, cnt, fp = c
            is_new = seg0 != cur_seg

            @pl.when(is_new)
            def _():
                flush(cur_seg, cnt, fp)
                reset_run()

            fp = fp + jnp.where(is_new, 1, 0).astype(jnp.int32)
            cnt = jnp.where(is_new, jnp.float32(0.0), cnt)

            def rbody(r, rc):
                out = []
                for j in range(NFC):
                    v = xbuf[b, r, pl.ds(j * LANES, LANES)]
                    out.append(rc[j] + v)
                for j in range(NFC):
                    v = xbuf[b, r, pl.ds(j * LANES, LANES)]
                    out.append(jnp.maximum(rc[NFC + j], v))
                return tuple(out)

            regs = lax.fori_loop(0, CHUNK, rbody, id_carry)
            for j in range(NFC):
                sl = pl.ds(j * LANES, LANES)
                plsc.addupdate(run.at[0, sl], regs[j])
                run[1, sl] = jnp.maximum(run[1, sl], regs[NFC + j])
            return (seg0, cnt + float(CHUNK), fp)

        def slow(c):
            def do_row(i, c):
                cur_seg, cnt, fp = c
                seg_i = bbuf[pl.ds(t0 + i, LANES)][0]
                is_new = seg_i != cur_seg

                @pl.when(is_new)
                def _():
                    flush(cur_seg, cnt, fp)
                    reset_run()

                fp = fp + jnp.where(is_new, 1, 0).astype(jnp.int32)
                cnt = jnp.where(is_new, jnp.float32(0.0), cnt)
                for j in range(NFC):
                    sl = pl.ds(j * LANES, LANES)
                    v = xbuf[b, i, sl]
                    plsc.addupdate(run.at[0, sl], v)
                    run[1, sl] = jnp.maximum(run[1, sl], v)
                return (seg_i, cnt + 1.0, fp)

            return lax.fori_loop(0, CHUNK, do_row, c)

        res = lax.cond(seg0 == seg_last, fast, slow, (cur_seg, cnt, fp))

        @pl.when(k + NBUF < nchunks)
        def _():
            start_x_copy(k + NBUF, b)

        return res

    reset_run()
    carry0 = (bbuf[pl.ds(0, LANES)][0], jnp.float32(0.0), jnp.int32(0))
    carry = lax.fori_loop(0, nchunks, process_chunk, carry0)

    cur_seg, cnt, fp = carry
    flush(cur_seg, cnt, fp)
    fp = fp + 1

    def drain(r, _):
        pltpu.make_async_copy(psum_hbm.at[0, pl.ds(0, 2), :], stage.at[0],
                              fsem).wait()
        return 0

    lax.fori_loop(0, jnp.minimum(fp, NSTAGE), drain, 0)
    pltpu.sync_copy(acc_c, pcnt_hbm.at[wid])


@functools.partial(
    pl.kernel,
    out_type=(
        jax.ShapeDtypeStruct((NW, N_SEG, N_FEAT), jnp.float32),
        jax.ShapeDtypeStruct((NW, N_SEG, N_FEAT), jnp.float32),
        jax.ShapeDtypeStruct((NW, N_SEG, LANES), jnp.float32),
    ),
    mesh=plsc.VectorSubcoreMesh(core_axis_name="c", subcore_axis_name="s"),
    scratch_types=[
        pltpu.VMEM((NBUF, CHUNK, N_FEAT), jnp.float32),
        pltpu.VMEM((RPT + 3 * LANES,), jnp.int32),
        pltpu.VMEM((2, N_FEAT), jnp.float32),
        pltpu.VMEM((NSTAGE, 2, N_FEAT), jnp.float32),
        pltpu.VMEM((N_SEG, LANES), jnp.float32),
        pltpu.SemaphoreType.DMA,
        pltpu.SemaphoreType.DMA,
    ],
)
def _sc_pool(x_hbm, batch_hbm, psum_hbm, pmax_hbm, pcnt_hbm,
             xbuf, bbuf, run, stage, acc_c, sem, fsem):
    _sc_pool_body(x_hbm, batch_hbm, psum_hbm, pmax_hbm, pcnt_hbm,
                  xbuf, bbuf, run, stage, acc_c, sem, fsem)


def _tc_merge_body(ps_ref, pm_ref, pc_ref, out_ref):
    valid = pc_ref[...][:, :, 0:1] > 0.0
    ps = jnp.where(valid, ps_ref[...], jnp.float32(0.0))
    pm = jnp.where(valid, pm_ref[...], NEG_INF)
    s = jnp.sum(ps, axis=0)
    m = jnp.max(pm, axis=0)
    c = jnp.sum(pc_ref[...], axis=0)[:, 0:1]
    mean = s / jnp.maximum(c, 1.0)
    mx = jnp.where(m == NEG_INF, jnp.float32(0.0), m)
    out_ref[...] = jnp.concatenate([mean, mx], axis=-1)


def _tc_merge(psum, pmax, pcnt):
    return pl.pallas_call(
        _tc_merge_body,
        out_shape=jax.ShapeDtypeStruct((N_SEG, 2 * N_FEAT), jnp.float32),
    )(psum, pmax, pcnt)


@jax.jit
def kernel(x, batch):
    batch32 = batch.astype(jnp.int32)
    psum, pmax, pcnt = _sc_pool(x, batch32)
    return _tc_merge(psum, pmax, pcnt)


# speedup vs baseline: 1.0083x; 1.0012x over previous
import functools

import jax
import jax.numpy as jnp
from jax import lax
from jax.experimental import pallas as pl
from jax.experimental.pallas import tpu as pltpu
from jax.experimental.pallas import tpu_sc as plsc

N_ROWS = 100000
N_FEAT = 256
N_SEG = 128
LANES = 16
NFC = N_FEAT // LANES
N_CORES = 2
N_SUBCORES = 16
NW = N_CORES * N_SUBCORES

RPT = 3136
LAST_ROWS = N_ROWS - (NW - 1) * RPT
CHUNK = 32
FULL_CHUNKS = RPT // CHUNK
LAST_CHUNKS = LAST_ROWS // CHUNK

NBUF = 8
NSTAGE = 8

NEG_INF = -float("inf")


def _sc_pool_body(x_hbm, batch_hbm, psum_hbm, pmax_hbm, pcnt_hbm,
                  xbuf, bbuf, run, stage, acc_c, sem, fsem):
    wid = lax.axis_index("s") * N_CORES + lax.axis_index("c")
    rbase = wid * RPT
    is_last = wid == NW - 1
    nchunks = jnp.where(is_last, LAST_CHUNKS, FULL_CHUNKS)
    nrows = jnp.where(is_last, LAST_ROWS, RPT)

    zeros = jnp.zeros((LANES,), jnp.float32)
    ninf = jnp.full((LANES,), NEG_INF, jnp.float32)

    def start_x_copy(k, b):
        pltpu.async_copy(x_hbm.at[pl.ds(rbase + k * CHUNK, CHUNK), :],
                         xbuf.at[b], sem)

    def wait_x_copy():
        pltpu.make_async_copy(x_hbm.at[pl.ds(0, CHUNK), :], xbuf.at[0],
                              sem).wait()

    for d in range(NBUF):
        @pl.when(d < nchunks)
        def _():
            start_x_copy(d, d)

    def copy_batch_full(_):
        pltpu.sync_copy(batch_hbm.at[pl.ds(rbase, RPT)], bbuf.at[pl.ds(0, RPT)])
        return 0

    def copy_batch_last(_):
        pltpu.sync_copy(batch_hbm.at[pl.ds(rbase, LAST_ROWS)],
                        bbuf.at[pl.ds(0, LAST_ROWS)])
        return 0

    lax.cond(is_last, copy_batch_last, copy_batch_full, 0)

    def init_cnt(r, _):
        acc_c[r, :] = zeros
        return 0

    lax.fori_loop(0, N_SEG, init_cnt, 0)

    id_carry = (zeros,) * NFC + (ninf,) * NFC

    def reset_run():
        for j in range(NFC):
            sl = pl.ds(j * LANES, LANES)
            run[0, sl] = zeros
            run[1, sl] = ninf

    def flush(cur_seg, cnt, fp):
        @pl.when(fp >= NSTAGE)
        def _():
            pltpu.make_async_copy(psum_hbm.at[0, pl.ds(0, 2), :],
                                  stage.at[0], fsem).wait()

        p = lax.rem(fp, NSTAGE)
        for j in range(NFC):
            sl = pl.ds(j * LANES, LANES)
            stage[p, 0, sl] = run[0, sl]
            stage[p, 1, sl] = run[1, sl]
        acc_c[cur_seg, :] = jnp.broadcast_to(cnt, (LANES,))
        pltpu.async_copy(stage.at[p, 0], psum_hbm.at[wid, cur_seg], fsem)
        pltpu.async_copy(stage.at[p, 1], pmax_hbm.at[wid, cur_seg], fsem)

    def process_chunk(k, carry):
        cur_seg, cnt, fp = carry
        wait_x_copy()
        b = lax.rem(k, NBUF)
        t0 = k * CHUNK
        seg0 = bbuf[pl.ds(t0, LANES)][0]
        seg_last = bbuf[pl.ds(t0 + CHUNK - LANES, LANES)][LANES - 1]

        def fast(c):
            cur_seg, cnt, fp = c
            is_new = seg0 != cur_seg

            @pl.when(is_new)
            def _():
                flush(cur_seg, cnt, fp)
                reset_run()

            fp = fp + jnp.where(is_new, 1, 0).astype(jnp.int32)
            cnt = jnp.where(is_new, jnp.float32(0.0), cnt)

            def rbody(r, rc):
                out = []
                for j in range(NFC):
                    v = xbuf[b, r, pl.ds(j * LANES, LANES)]
                    out.append(rc[j] + v)
                for j in range(NFC):
                    v = xbuf[b, r, pl.ds(j * LANES, LANES)]
                    out.append(jnp.maximum(rc[NFC + j], v))
                return tuple(out)

            regs = lax.fori_loop(0, CHUNK, rbody, id_carry)
            for j in range(NFC):
                sl = pl.ds(j * LANES, LANES)
                plsc.addupdate(run.at[0, sl], regs[j])
                run[1, sl] = jnp.maximum(run[1, sl], regs[NFC + j])
            return (seg0, cnt + float(CHUNK), fp)

        def slow(c):
            def do_row(i, c):
                cur_seg, cnt, fp = c
                seg_i = bbuf[pl.ds(t0 + i, LANES)][0]
                is_new = seg_i != cur_seg

                @pl.when(is_new)
                def _():
                    flush(cur_seg, cnt, fp)
                    reset_run()

                fp = fp + jnp.where(is_new, 1, 0).astype(jnp.int32)
                cnt = jnp.where(is_new, jnp.float32(0.0), cnt)
                for j in range(NFC):
                    sl = pl.ds(j * LANES, LANES)
                    v = xbuf[b, i, sl]
                    plsc.addupdate(run.at[0, sl], v)
                    run[1, sl] = jnp.maximum(run[1, sl], v)
                return (seg_i, cnt + 1.0, fp)

            return lax.fori_loop(0, CHUNK, do_row, c)

        res = lax.cond(seg0 == seg_last, fast, slow, (cur_seg, cnt, fp))

        @pl.when(k + NBUF < nchunks)
        def _():
            start_x_copy(k + NBUF, b)

        return res

    reset_run()
    carry0 = (bbuf[pl.ds(0, LANES)][0], jnp.float32(0.0), jnp.int32(0))
    carry = lax.fori_loop(0, nchunks, process_chunk, carry0)

    cur_seg, cnt, fp = carry
    flush(cur_seg, cnt, fp)
    fp = fp + 1

    def drain(r, _):
        pltpu.make_async_copy(psum_hbm.at[0, pl.ds(0, 2), :], stage.at[0],
                              fsem).wait()
        return 0

    lax.fori_loop(0, jnp.minimum(fp, NSTAGE), drain, 0)
    pltpu.sync_copy(acc_c, pcnt_hbm.at[wid])


@functools.partial(
    pl.kernel,
    out_type=(
        jax.ShapeDtypeStruct((NW, N_SEG, N_FEAT), jnp.float32),
        jax.ShapeDtypeStruct((NW, N_SEG, N_FEAT), jnp.float32),
        jax.ShapeDtypeStruct((NW, N_SEG, LANES), jnp.float32),
    ),
    mesh=plsc.VectorSubcoreMesh(core_axis_name="c", subcore_axis_name="s"),
    scratch_types=[
        pltpu.VMEM((NBUF, CHUNK, N_FEAT), jnp.float32),
        pltpu.VMEM((RPT + 3 * LANES,), jnp.int32),
        pltpu.VMEM((2, N_FEAT), jnp.float32),
        pltpu.VMEM((NSTAGE, 2, N_FEAT), jnp.float32),
        pltpu.VMEM((N_SEG, LANES), jnp.float32),
        pltpu.SemaphoreType.DMA,
        pltpu.SemaphoreType.DMA,
    ],
)
def _sc_pool(x_hbm, batch_hbm, psum_hbm, pmax_hbm, pcnt_hbm,
             xbuf, bbuf, run, stage, acc_c, sem, fsem):
    _sc_pool_body(x_hbm, batch_hbm, psum_hbm, pmax_hbm, pcnt_hbm,
                  xbuf, bbuf, run, stage, acc_c, sem, fsem)


def _tc_merge_body(ps_ref, pm_ref, pc_ref, out_ref):
    valid = pc_ref[...][:, :, 0:1] > 0.0
    ps = jnp.where(valid, ps_ref[...], jnp.float32(0.0))
    pm = jnp.where(valid, pm_ref[...], NEG_INF)
    s = jnp.sum(ps, axis=0)
    m = jnp.max(pm, axis=0)
    c = jnp.sum(pc_ref[...], axis=0)[:, 0:1]
    mean = s / jnp.maximum(c, 1.0)
    mx = jnp.where(m == NEG_INF, jnp.float32(0.0), m)
    out_ref[...] = jnp.concatenate([mean, mx], axis=-1)


def _tc_merge(psum, pmax, pcnt):
    return pl.pallas_call(
        _tc_merge_body,
        out_shape=jax.ShapeDtypeStruct((N_SEG, 2 * N_FEAT), jnp.float32),
    )(psum, pmax, pcnt)


@jax.jit
def kernel(x, batch):
    batch32 = batch.astype(jnp.int32)
    psum, pmax, pcnt = _sc_pool(x, batch32)
    return _tc_merge(psum, pmax, pcnt)


# speedup vs baseline: 1.2188x; 1.2087x over previous
import functools

import jax
import jax.numpy as jnp
from jax import lax
from jax.experimental import pallas as pl
from jax.experimental.pallas import tpu as pltpu
from jax.experimental.pallas import tpu_sc as plsc

N_ROWS = 100000
N_FEAT = 256
N_SEG = 128
LANES = 16
NFC = N_FEAT // LANES
N_CORES = 2
N_SUBCORES = 16
NW = N_CORES * N_SUBCORES

RPT = 3136
LAST_ROWS = N_ROWS - (NW - 1) * RPT
CHUNK = 32
FULL_CHUNKS = RPT // CHUNK
LAST_CHUNKS = LAST_ROWS // CHUNK

NEG_INF = -float("inf")


NBUF = 5


def _sc_pool_body(x_hbm, batch_hbm, psum_hbm, pmax_hbm, pcnt_hbm,
                  xbuf, bbuf, acc_s, acc_m, acc_c, sem):
    wid = lax.axis_index("s") * N_CORES + lax.axis_index("c")
    rbase = wid * RPT
    is_last = wid == NW - 1
    nchunks = jnp.where(is_last, LAST_CHUNKS, FULL_CHUNKS)

    zeros = jnp.zeros((LANES,), jnp.float32)
    ninf = jnp.full((LANES,), NEG_INF, jnp.float32)
    ones = jnp.ones((LANES,), jnp.float32)
    chunk_f = jnp.full((LANES,), float(CHUNK), jnp.float32)

    def start_x_copy(k, b):
        pltpu.async_copy(x_hbm.at[pl.ds(rbase + k * CHUNK, CHUNK), :],
                         xbuf.at[b], sem)

    def wait_x_copy():
        pltpu.make_async_copy(x_hbm.at[pl.ds(0, CHUNK), :], xbuf.at[0],
                              sem).wait()

    for d in range(NBUF):
        @pl.when(d < nchunks)
        def _():
            start_x_copy(d, d)

    def copy_batch_full(_):
        pltpu.sync_copy(batch_hbm.at[pl.ds(rbase, RPT)], bbuf.at[pl.ds(0, RPT)])
        return 0

    def copy_batch_last(_):
        pltpu.sync_copy(batch_hbm.at[pl.ds(rbase, LAST_ROWS)],
                        bbuf.at[pl.ds(0, LAST_ROWS)])
        return 0

    lax.cond(is_last, copy_batch_last, copy_batch_full, 0)

    nrows = jnp.where(is_last, LAST_ROWS, RPT)
    seg_lo = bbuf[pl.ds(0, LANES)][0]
    seg_hi = bbuf[pl.ds(nrows - LANES, LANES)][LANES - 1]

    def init_cnt(r, _):
        acc_c[r, :] = zeros
        return 0

    lax.fori_loop(0, N_SEG, init_cnt, 0)

    def init_row(r, _):
        for j in range(NFC):
            sl = pl.ds(j * LANES, LANES)
            acc_s[r, sl] = zeros
            acc_m[r, sl] = ninf
        return 0

    lax.fori_loop(seg_lo, seg_hi + 1, init_row, 0)

    def process_chunk(b, t0):
        seg0 = bbuf[pl.ds(t0, LANES)][0]
        seg_last = bbuf[pl.ds(t0 + CHUNK - LANES, LANES)][LANES - 1]

        def fast(_):
            def rbody(r, carry):
                out = []
                for j in range(NFC):
                    v = xbuf[b, r, pl.ds(j * LANES, LANES)]
                    out.append(carry[j] + v)
                for j in range(NFC):
                    v = xbuf[b, r, pl.ds(j * LANES, LANES)]
                    out.append(jnp.maximum(carry[NFC + j], v))
                return tuple(out)

            carry0 = (zeros,) * NFC + (ninf,) * NFC
            carry = lax.fori_loop(0, CHUNK, rbody, carry0)
            for j in range(NFC):
                sl = pl.ds(j * LANES, LANES)
                plsc.addupdate(acc_s.at[seg0, sl], carry[j])
                acc_m[seg0, sl] = jnp.maximum(acc_m[seg0, sl], carry[NFC + j])
            plsc.addupdate(acc_c.at[seg0], chunk_f)
            return 0

        def slow(_):
            def do_row(i, _):
                seg = bbuf[pl.ds(t0 + i, LANES)][0]
                for j in range(NFC):
                    sl = pl.ds(j * LANES, LANES)
                    v = xbuf[b, i, sl]
                    plsc.addupdate(acc_s.at[seg, sl], v)
                    acc_m[seg, sl] = jnp.maximum(acc_m[seg, sl], v)
                plsc.addupdate(acc_c.at[seg], ones)
                return 0

            lax.fori_loop(0, CHUNK, do_row, 0)
            return 0

        lax.cond(seg0 == seg_last, fast, slow, 0)

    def chunk_body(k, _):
        wait_x_copy()
        b = lax.rem(k, NBUF)
        process_chunk(b, k * CHUNK)

        @pl.when(k + NBUF < nchunks)
        def _():
            start_x_copy(k + NBUF, lax.rem(k + NBUF, NBUF))

        return 0

    lax.fori_loop(0, nchunks, chunk_body, 0)

    def write_row(r, _):
        pltpu.async_copy(acc_s.at[r], psum_hbm.at[wid, r], sem)
        pltpu.async_copy(acc_m.at[r], pmax_hbm.at[wid, r], sem)
        return 0

    lax.fori_loop(seg_lo, seg_hi + 1, write_row, 0)
    pltpu.sync_copy(acc_c, pcnt_hbm.at[wid])

    def drain_row(r, _):
        pltpu.make_async_copy(acc_s.at[0], psum_hbm.at[wid, 0], sem).wait()
        pltpu.make_async_copy(acc_m.at[0], pmax_hbm.at[wid, 0], sem).wait()
        return 0

    lax.fori_loop(seg_lo, seg_hi + 1, drain_row, 0)


@functools.partial(
    pl.kernel,
    out_type=(
        jax.ShapeDtypeStruct((NW, N_SEG, N_FEAT), jnp.float32),
        jax.ShapeDtypeStruct((NW, N_SEG, N_FEAT), jnp.float32),
        jax.ShapeDtypeStruct((NW, N_SEG, LANES), jnp.float32),
    ),
    mesh=plsc.VectorSubcoreMesh(core_axis_name="c", subcore_axis_name="s"),
    scratch_types=[
        pltpu.VMEM((NBUF, CHUNK, N_FEAT), jnp.float32),
        pltpu.VMEM((RPT + LANES,), jnp.int32),
        pltpu.VMEM((N_SEG, N_FEAT), jnp.float32),
        pltpu.VMEM((N_SEG, N_FEAT), jnp.float32),
        pltpu.VMEM((N_SEG, LANES), jnp.float32),
        pltpu.SemaphoreType.DMA,
    ],
)
def _sc_pool(x_hbm, batch_hbm, psum_hbm, pmax_hbm, pcnt_hbm,
             xbuf, bbuf, acc_s, acc_m, acc_c, sem):
    _sc_pool_body(x_hbm, batch_hbm, psum_hbm, pmax_hbm, pcnt_hbm,
                  xbuf, bbuf, acc_s, acc_m, acc_c, sem)


def _tc_merge_body(ps_ref, pm_ref, pc_ref, out_ref):
    valid = pc_ref[...][:, :, 0:1] > 0.0
    ps = jnp.where(valid, ps_ref[...], jnp.float32(0.0))
    pm = jnp.where(valid, pm_ref[...], NEG_INF)
    s = jnp.sum(ps, axis=0)
    m = jnp.max(pm, axis=0)
    c = jnp.sum(pc_ref[...], axis=0)[:, 0:1]
    mean = s / jnp.maximum(c, 1.0)
    mx = jnp.where(m == NEG_INF, jnp.float32(0.0), m)
    out_ref[...] = jnp.concatenate([mean, mx], axis=-1)


def _tc_merge(psum, pmax, pcnt):
    return pl.pallas_call(
        _tc_merge_body,
        out_shape=jax.ShapeDtypeStruct((N_SEG, 2 * N_FEAT), jnp.float32),
    )(psum, pmax, pcnt)


@jax.jit
def kernel(x, batch):
    batch32 = batch.astype(jnp.int32)
    psum, pmax, pcnt = _sc_pool(x, batch32)
    return _tc_merge(psum, pmax, pcnt)
